# Initial kernel scaffold; baseline (speedup 1.0000x reference)
import jax
import jax.numpy as jnp
from jax.experimental import pallas as pl


def kernel(x, edge_index, batch, W1, as1, ad1, b1, g1, be1, W2, as2, ad2, b2, g2, be2, W3, as3, ad3, b3, g3, be3, W4, as4, ad4, b4, g4, be4, fcW, fcb):
    raise NotImplementedError("write your pallas kernel here")


# speedup vs baseline: 9.1863x; 9.1863x over previous
import functools

import jax
import jax.numpy as jnp
from jax import lax
from jax.experimental import pallas as pl
from jax.experimental.pallas import tpu as pltpu
from jax.experimental.pallas import tpu_sc as plsc

N = 10000
D = 128
H = 128
HH = H // 2
NG = 64
NP = 10240
NP2 = NP // 2
SR = NP // H
CH = 128
NSUB = 16
TPT = 20736
NCH = TPT // CH
EP = NSUB * TPT
APT = NP2 // NSUB
DUMMY = N + 200
F32 = jnp.float32


def _write_table(h, asv, adv, table_ref, hs_ref, hd_ref):
    table_ref[...] = h
    hs_ref[...] = jnp.dot(h, asv, preferred_element_type=F32,
                          precision=lax.Precision.HIGHEST)
    hd_ref[...] = jnp.dot(h, adv, preferred_element_type=F32,
                          precision=lax.Precision.HIGHEST)


def _entry_body(x_ref, w_ref, asv_ref, adv_ref, table_ref, hs_ref, hd_ref):
    h = jnp.dot(x_ref[...], w_ref[...], preferred_element_type=F32,
                precision=lax.Precision.HIGHEST)
    _write_table(h, asv_ref[...], adv_ref[...], table_ref, hs_ref, hd_ref)


def _make_entry():
    return pl.pallas_call(
        _entry_body,
        grid=(NBLK,),
        out_shape=(_SDS((NP, H), F32), _SDS((NP, 1), F32),
                   _SDS((NP, 1), F32)),
        in_specs=[pl.BlockSpec((BLK, H), lambda i: (i, 0)),
                  pl.BlockSpec((H, H), lambda i: (0, 0)),
                  _VEC_SPEC, _VEC_SPEC],
        out_specs=(pl.BlockSpec((BLK, H), lambda i: (i, 0)),
                   pl.BlockSpec((BLK, 1), lambda i: (i, 0)),
                   pl.BlockSpec((BLK, 1), lambda i: (i, 0))),
    )


BLK = 2048
NBLK = NP // BLK


def _block_y(part_ref, s_ref, b_ref, i):
    accs = jnp.concatenate([part_ref[0], part_ref[1]], axis=1)
    y = accs / (s_ref[0] + 1e-16) + b_ref[...]
    valid = ((lax.broadcasted_iota(jnp.int32, (BLK, 1), 0) + i * BLK)
             < N).astype(F32)
    return y, valid


def _stats_body(part_ref, s_ref, b_ref, stats_ref):
    i = pl.program_id(0)
    y, valid = _block_y(part_ref, s_ref, b_ref, i)
    yv = y * valid
    blk = jnp.concatenate(
        [jnp.sum(yv, axis=0, keepdims=True),
         jnp.sum(yv * y, axis=0, keepdims=True)], axis=0)

    @pl.when(i == 0)
    def _init():
        stats_ref[...] = blk

    @pl.when(i != 0)
    def _accum():
        stats_ref[...] += blk


def _block_z(part_ref, s_ref, stats_ref, b_ref, g_ref, be_ref, i):
    y, valid = _block_y(part_ref, s_ref, b_ref, i)
    mu = stats_ref[0:1] * (1.0 / N)
    var = stats_ref[1:2] * (1.0 / N) - mu * mu
    z = (y - mu) * lax.rsqrt(var + 1e-5) * g_ref[...] + be_ref[...]
    return jnp.maximum(z, 0.0) * valid, valid


def _apply_body(part_ref, s_ref, stats_ref, b_ref, g_ref, be_ref, w_ref,
                asv_ref, adv_ref, table_ref, hs_ref, hd_ref):
    i = pl.program_id(0)
    zv, _ = _block_z(part_ref, s_ref, stats_ref, b_ref, g_ref, be_ref, i)
    h = jnp.dot(zv, w_ref[...], preferred_element_type=F32,
                precision=lax.Precision.HIGHEST)
    _write_table(h, asv_ref[...], adv_ref[...], table_ref, hs_ref, hd_ref)


def _pool_body(part_ref, s_ref, stats_ref, b_ref, g_ref, be_ref, batch_ref,
               fcw_ref, fcb_ref, out_ref, sums_ref, counts_ref):
    i = pl.program_id(0)
    zv, valid = _block_z(part_ref, s_ref, stats_ref, b_ref, g_ref, be_ref, i)
    groups = lax.broadcasted_iota(jnp.int32, (BLK, NG), 1)
    onehot = jnp.where(batch_ref[...] == groups, 1.0, 0.0).astype(F32) * valid
    bsums = lax.dot_general(onehot, zv, (((0,), (0,)), ((), ())),
                            preferred_element_type=F32,
                            precision=lax.Precision.HIGHEST)
    bcounts = lax.dot_general(onehot, valid, (((0,), (0,)), ((), ())),
                              preferred_element_type=F32,
                              precision=lax.Precision.HIGHEST)

    @pl.when(i == 0)
    def _init():
        sums_ref[...] = bsums
        counts_ref[...] = bcounts

    @pl.when(i != 0)
    def _accum():
        sums_ref[...] += bsums
        counts_ref[...] += bcounts

    @pl.when(i == NBLK - 1)
    def _head():
        pooled = sums_ref[...] / jnp.maximum(counts_ref[...], 1.0)
        out_ref[...] = jnp.dot(pooled, fcw_ref[...],
                               preferred_element_type=F32,
                               precision=lax.Precision.HIGHEST) + fcb_ref[...]


def _tc_call(body, n_in, out_shapes):
    multi = isinstance(out_shapes, tuple)
    return pl.pallas_call(
        body,
        out_shape=out_shapes,
        in_specs=[pl.BlockSpec(memory_space=pltpu.VMEM)] * n_in,
        out_specs=(tuple(pl.BlockSpec(memory_space=pltpu.VMEM)
                         for _ in out_shapes)
                   if multi else pl.BlockSpec(memory_space=pltpu.VMEM)),
    )


_SDS = jax.ShapeDtypeStruct

_PART_SPEC = pl.BlockSpec((2, BLK, HH), lambda i: (0, i, 0))
_S_SPEC = pl.BlockSpec((2, BLK, 1), lambda i: (0, i, 0))
_ROW_SPEC = pl.BlockSpec((1, H), lambda i: (0, 0))
_VEC_SPEC = pl.BlockSpec((H, 1), lambda i: (0, 0))


def _make_stats():
    return pl.pallas_call(
        _stats_body,
        grid=(NBLK,),
        out_shape=_SDS((2, H), F32),
        in_specs=[_PART_SPEC, _S_SPEC, _ROW_SPEC],
        out_specs=pl.BlockSpec((2, H), lambda i: (0, 0)),
    )


def _make_apply():
    return pl.pallas_call(
        _apply_body,
        grid=(NBLK,),
        out_shape=(_SDS((NP, H), F32), _SDS((NP, 1), F32),
                   _SDS((NP, 1), F32)),
        in_specs=[_PART_SPEC, _S_SPEC,
                  pl.BlockSpec((2, H), lambda i: (0, 0)),
                  _ROW_SPEC, _ROW_SPEC, _ROW_SPEC,
                  pl.BlockSpec((H, H), lambda i: (0, 0)),
                  _VEC_SPEC, _VEC_SPEC],
        out_specs=(pl.BlockSpec((BLK, H), lambda i: (i, 0)),
                   pl.BlockSpec((BLK, 1), lambda i: (i, 0)),
                   pl.BlockSpec((BLK, 1), lambda i: (i, 0))),
    )


def _make_pool():
    return pl.pallas_call(
        _pool_body,
        grid=(NBLK,),
        out_shape=(_SDS((NG, 1), F32), _SDS((NG, H), F32),
                   _SDS((NG, 1), F32)),
        in_specs=[_PART_SPEC, _S_SPEC,
                  pl.BlockSpec((2, H), lambda i: (0, 0)),
                  _ROW_SPEC, _ROW_SPEC, _ROW_SPEC,
                  pl.BlockSpec((BLK, 1), lambda i: (i, 0)),
                  _VEC_SPEC, pl.BlockSpec((1, 1), lambda i: (0, 0))],
        out_specs=(pl.BlockSpec((NG, 1), lambda i: (0, 0)),
                   pl.BlockSpec((NG, H), lambda i: (0, 0)),
                   pl.BlockSpec((NG, 1), lambda i: (0, 0))),
    )


def _edge_body(table_hbm, hs_hbm, hd_hbm, src_hbm, dst_hbm,
               out_hbm, outs_hbm,
               hs_v, hd_v, srcb, dstb, rows, srow, exb, dhf, dstb2,
               sloc, sidx, acc, sacc, sem):
    c = lax.axis_index("c")
    s = lax.axis_index("s")

    def _zero_rows(j, carry):
        for g2 in range(H // 16):
            rows[0, j, pl.ds(g2 * 16, 16)] = jnp.zeros((16,), F32)
            srow[j, pl.ds(g2 * 16, 16)] = jnp.zeros((16,), F32)
        return carry

    lax.fori_loop(0, CH, _zero_rows, 0)

    def _zero_sloc(j, carry):
        for g2 in range(H // 16):
            sloc[j, pl.ds(g2 * 16, 16)] = jnp.zeros((16,), F32)
        return carry

    lax.fori_loop(0, SR, _zero_sloc, 0)

    for i in range(SR // 16):
        sidx[pl.ds(i * 16, 16)] = (
            lax.broadcasted_iota(jnp.int32, (16,), 0) + (i * 16))

    for i in range(APT // CH + 1):
        sz = CH if (i + 1) * CH <= APT else APT - (APT // CH) * CH
        pltpu.sync_copy(rows.at[0, pl.ds(0, sz)],
                        acc.at[pl.ds(s * APT + i * CH, sz)])

    @pl.when(s < SR // 8)
    def _init_sacc():
        pltpu.sync_copy(sloc.at[pl.ds(0, 8)], sacc.at[pl.ds(s * 8, 8)])

    plsc.subcore_barrier()

    pltpu.sync_copy(hs_hbm, hs_v)
    pltpu.sync_copy(hd_hbm, hd_v)

    base = s * TPT

    def _chunk(k, carry):
        off = base + k * CH
        pltpu.sync_copy(src_hbm.at[pl.ds(off, CH)], srcb.at[0])
        pltpu.sync_copy(dst_hbm.at[pl.ds(off, CH)], dstb.at[0])
        pltpu.async_copy(table_hbm.at[srcb.at[0]], rows.at[0], sem).wait()
        for i in range(CH // 16):
            s16 = srcb[0, pl.ds(i * 16, 16)]
            d16 = dstb[0, pl.ds(i * 16, 16)]
            t = plsc.load_gather(hs_v, [s16]) + plsc.load_gather(hd_v, [d16])
            ex = jnp.exp(jnp.maximum(t, 0.2 * t))
            exb[pl.ds(i * 16, 16)] = ex
            dhf[pl.ds(i * 16, 16)] = lax.convert_element_type(
                lax.bitwise_and(d16, 1), F32)
            dstb2[pl.ds(i * 16, 16)] = lax.shift_right_logical(d16, 1)
            plsc.addupdate_scatter(
                sloc, [lax.shift_right_logical(d16, 7),
                       lax.bitwise_and(d16, 127)], ex)

        def _scale(j, c2):
            ev = plsc.load_gather(exb, [lax.broadcast(j, (16,))])
            dh = plsc.load_gather(dhf, [lax.broadcast(j, (16,))])
            for g2 in range(HH // 16):
                v = rows[0, j, pl.ds(c * HH + g2 * 16, 16)] * ev
                srow[j, pl.ds(g2 * 16, 16)] = v * (1.0 - dh)
                srow[j, pl.ds(HH + g2 * 16, 16)] = v * dh
            return c2

        lax.fori_loop(0, CH, _scale, 0)
        pltpu.sync_copy(srow, acc.at[dstb2], add=True)
        return carry

    lax.fori_loop(0, NCH, _chunk, 0)
    pltpu.sync_copy(sloc, sacc.at[sidx], add=True)
    plsc.subcore_barrier()

    for i in range(APT // CH + 1):
        sz = CH if (i + 1) * CH <= APT else APT - (APT // CH) * CH
        pltpu.sync_copy(acc.at[pl.ds(s * APT + i * CH, sz)],
                        rows.at[0, pl.ds(0, sz)])
        pltpu.sync_copy(rows.at[0, pl.ds(0, sz)],
                        out_hbm.at[c, pl.ds(s * APT + i * CH, sz)])

    @pl.when(s < SR // 8)
    def _write_sacc():
        pltpu.sync_copy(sacc.at[pl.ds(s * 8, 8)], sloc.at[pl.ds(0, 8)])
        pltpu.sync_copy(sloc.at[pl.ds(0, 8)], outs_hbm.at[c, pl.ds(s * 8, 8)])


@functools.cache
def _make_edge_kernel():
    mesh = plsc.VectorSubcoreMesh(core_axis_name="c", subcore_axis_name="s")
    return pl.kernel(
        _edge_body,
        out_type=(_SDS((2, NP2, H), F32), _SDS((2, SR, H), F32)),
        mesh=mesh,
        scratch_types=[
            pltpu.VMEM((NP,), F32),
            pltpu.VMEM((NP,), F32),
            pltpu.VMEM((2, CH), jnp.int32),
            pltpu.VMEM((2, CH), jnp.int32),
            pltpu.VMEM((2, CH, H), F32),
            pltpu.VMEM((CH, H), F32),
            pltpu.VMEM((CH,), F32),
            pltpu.VMEM((CH,), F32),
            pltpu.VMEM((CH,), jnp.int32),
            pltpu.VMEM((SR, H), F32),
            pltpu.VMEM((SR,), jnp.int32),
            pltpu.VMEM_SHARED((NP2, H), F32),
            pltpu.VMEM_SHARED((SR, H), F32),
            pltpu.SemaphoreType.DMA,
        ],
        compiler_params=pltpu.CompilerParams(needs_layout_passes=False),
    )


def kernel(x, edge_index, batch, W1, as1, ad1, b1, g1, be1, W2, as2, ad2, b2,
           g2, be2, W3, as3, ad3, b3, g3, be3, W4, as4, ad4, b4, g4, be4,
           fcW, fcb):
    loop = jnp.arange(N, dtype=jnp.int32)
    npad = EP - (edge_index.shape[1] + N)
    src = jnp.concatenate([edge_index[0].astype(jnp.int32), loop,
                           jnp.zeros((npad,), jnp.int32)])
    dst = jnp.concatenate([edge_index[1].astype(jnp.int32), loop,
                           jnp.full((npad,), DUMMY, jnp.int32)])
    xp = jnp.pad(x, ((0, NP - N), (0, 0)))
    batch_p = jnp.pad(batch.astype(jnp.int32), (0, NP - N)).reshape(NP, 1)

    entry = _make_entry()
    stats = _make_stats()
    apply_ = _make_apply()
    pool = _make_pool()
    edge = _make_edge_kernel()

    table, hs, hd = entry(xp, W1, as1.reshape(H, 1), ad1.reshape(H, 1))
    layers = [(W2, as2, ad2, b1, g1, be1), (W3, as3, ad3, b2, g2, be2),
              (W4, as4, ad4, b3, g3, be3)]
    for (w, a_s, a_d, b, g, be) in layers:
        part, spart = edge(table, hs.reshape(NP), hd.reshape(NP), src, dst)
        pr = part.reshape(2, NP, HH)
        sr = spart.reshape(2, NP, 1)
        st = stats(pr, sr, b.reshape(1, H))
        table, hs, hd = apply_(pr, sr, st, b.reshape(1, H), g.reshape(1, H),
                               be.reshape(1, H), w, a_s.reshape(H, 1),
                               a_d.reshape(H, 1))
    part, spart = edge(table, hs.reshape(NP), hd.reshape(NP), src, dst)
    pr = part.reshape(2, NP, HH)
    sr = spart.reshape(2, NP, 1)
    st = stats(pr, sr, b4.reshape(1, H))
    out, _, _ = pool(pr, sr, st, b4.reshape(1, H), g4.reshape(1, H),
                     be4.reshape(1, H), batch_p, fcW, fcb.reshape(1, 1))
    return out
